# Initial kernel scaffold; baseline (speedup 1.0000x reference)
import jax
import jax.numpy as jnp
from jax.experimental import pallas as pl


def kernel(x, edge_index, edge_weight, W1, b1, W2, b2):
    raise NotImplementedError("write your pallas kernel here")


# speedup vs baseline: 2.5081x; 2.5081x over previous
import functools

import jax
import jax.numpy as jnp
from jax import lax
from jax.experimental import pallas as pl
from jax.experimental.pallas import tpu as pltpu
from jax.experimental.pallas import tpu_sc as plsc

N_NODES = 10000
N_PAD = 10240
NFEAT = 128
NHID = 256
NCLASS = 64

NC = 2
NS = 16
L = 16

K = 512
B = 128
NB = K // B
E_PAD = 327680
EPW = E_PAD // NS
N_CHUNKS = EPW // K
ROWS_PER_TILE = N_PAD // NS
DROWS = EPW // B


def _sc_mp(sup_flat, src1d, dst2d, w1d, nsplit, dh):
    npass = nsplit // NC
    mesh = plsc.VectorSubcoreMesh(core_axis_name="c", subcore_axis_name="s",
                                  num_cores=NC, num_subcores=NS)

    @functools.partial(
        pl.kernel,
        out_type=jax.ShapeDtypeStruct((nsplit, N_PAD, dh), jnp.float32),
        mesh=mesh,
        compiler_params=pltpu.CompilerParams(use_tc_tiling_on_sc=False,
                                             needs_layout_passes=False),
        scratch_types=[
            pltpu.VMEM((K, dh), jnp.float32),
            pltpu.VMEM((K,), jnp.int32),
            pltpu.VMEM((DROWS, B), jnp.int32),
            pltpu.VMEM((K,), jnp.float32),
            pltpu.VMEM_SHARED((N_PAD, dh), jnp.float32),
            pltpu.SemaphoreType.DMA,
        ],
    )
    def mp(sup_hbm, src_hbm, dst_hbm, w_hbm, out_hbm, rows_v, src_v, dst_v, w_v, acc_s, sem):
        c = lax.axis_index("c")
        s = lax.axis_index("s")
        zero = jnp.zeros((L,), jnp.float32)
        ebase = s * EPW

        pltpu.sync_copy(dst_hbm.at[pl.ds(s * DROWS, DROWS)], dst_v)

        for q in range(npass):
            qq = q * NC + c

            def zbody(i, _):
                for f in range(dh // L):
                    rows_v[i, pl.ds(f * L, L)] = zero
                return 0
            lax.fori_loop(0, K, zbody, 0)
            pltpu.sync_copy(rows_v, acc_s.at[pl.ds(s * ROWS_PER_TILE, K)])
            pltpu.sync_copy(rows_v.at[pl.ds(0, ROWS_PER_TILE - K)],
                            acc_s.at[pl.ds(s * ROWS_PER_TILE + K, ROWS_PER_TILE - K)])
            plsc.subcore_barrier()

            def chunk(i, _):
                pltpu.sync_copy(src_hbm.at[pl.ds(ebase + i * K, K)], src_v)
                pltpu.sync_copy(w_hbm.at[pl.ds(ebase + i * K, K)], w_v)

                def ib(j, _):
                    v = src_v[pl.ds(j * L, L)]
                    src_v[pl.ds(j * L, L)] = v * nsplit + qq
                    return 0
                lax.fori_loop(0, K // L, ib, 0)

                cps = [pltpu.async_copy(sup_hbm.at[src_v.at[pl.ds(j * B, B)]],
                                        rows_v.at[pl.ds(j * B, B)], sem)
                       for j in range(NB)]
                for cp in cps:
                    cp.wait()

                def sb(e, _):
                    wv = plsc.load_gather(w_v, [jnp.full((L,), e, jnp.int32)])
                    for f in range(dh // L):
                        rows_v[e, pl.ds(f * L, L)] = rows_v[e, pl.ds(f * L, L)] * wv
                    return 0
                lax.fori_loop(0, K, sb, 0)

                for j in range(NB):
                    pltpu.sync_copy(rows_v.at[pl.ds(j * B, B)],
                                    acc_s.at[dst_v.at[i * NB + j]], add=True)
                return 0
            lax.fori_loop(0, N_CHUNKS, chunk, 0)
            plsc.subcore_barrier()

            pltpu.sync_copy(acc_s.at[pl.ds(s * ROWS_PER_TILE, ROWS_PER_TILE)],
                            out_hbm.at[qq, pl.ds(s * ROWS_PER_TILE, ROWS_PER_TILE)])
            plsc.subcore_barrier()

    return mp(sup_flat, src1d, dst2d, w1d)


def _mm1_body(x_ref, w_ref, o_ref):
    o_ref[...] = jnp.dot(x_ref[...], w_ref[...], preferred_element_type=jnp.float32)


def _mm2_body(a_ref, b_ref, w_ref, o_ref):
    acc = None
    for j in range(a_ref.shape[0]):
        h = jax.nn.relu(a_ref[j] + b_ref[j])
        t = jnp.dot(h, w_ref[j], preferred_element_type=jnp.float32)
        acc = t if acc is None else acc + t
    o_ref[...] = acc


def _fin_body(a_ref, b_ref, o_ref):
    z = jnp.concatenate([a_ref[j] for j in range(a_ref.shape[0])], axis=1) + b_ref[...]
    m = jnp.max(z, axis=1, keepdims=True)
    zs = z - m
    lse = jnp.log(jnp.sum(jnp.exp(zs), axis=1, keepdims=True))
    o_ref[...] = zs - lse


def kernel(x, edge_index, edge_weight, W1, b1, W2, b2):
    n_edges = edge_index.shape[1]
    src = edge_index[0].astype(jnp.int32)
    dst = edge_index[1].astype(jnp.int32)
    pad = E_PAD - n_edges
    src1d = jnp.concatenate([src, jnp.zeros((pad,), jnp.int32)])
    dst2d = jnp.concatenate([dst, jnp.zeros((pad,), jnp.int32)]).reshape(E_PAD // B, B)
    w1d = jnp.concatenate([edge_weight, jnp.zeros((pad,), jnp.float32)])

    support1 = pl.pallas_call(
        _mm1_body,
        out_shape=jax.ShapeDtypeStruct((N_NODES, NHID), jnp.float32),
    )(x, W1)

    ns1 = 4
    d1 = NHID // ns1
    agg1 = _sc_mp(support1.reshape(ns1 * N_NODES, d1), src1d, dst2d, w1d, ns1, d1)

    support2 = pl.pallas_call(
        _mm2_body,
        out_shape=jax.ShapeDtypeStruct((N_PAD, NCLASS), jnp.float32),
    )(agg1, b1.reshape(ns1, 1, d1), W2.reshape(ns1, d1, NCLASS))

    ns2 = 2
    d2 = NCLASS // ns2
    agg2 = _sc_mp(support2.reshape(ns2 * N_PAD, d2), src1d, dst2d, w1d, ns2, d2)

    out = pl.pallas_call(
        _fin_body,
        out_shape=jax.ShapeDtypeStruct((N_PAD, NCLASS), jnp.float32),
    )(agg2, b2.reshape(1, NCLASS))
    return out[:N_NODES]


# speedup vs baseline: 3.5720x; 1.4242x over previous
import functools

import jax
import jax.numpy as jnp
from jax import lax
from jax.experimental import pallas as pl
from jax.experimental.pallas import tpu as pltpu
from jax.experimental.pallas import tpu_sc as plsc

N_NODES = 10000
N_PAD = 10240
NFEAT = 128
NHID = 256
NCLASS = 64

NC = 2
NS = 16
L = 16

K = 512
B = 128
NB = K // B
NRB = 3
NLB = 4
E_PAD = 327680
EPW = E_PAD // NS
NCH = EPW // K
ROWS_PER_TILE = N_PAD // NS
DROWS = EPW // B


def _sc_mp(sup_flat, src1d, dst2d, w1d, nsplit, dh):
    npass = nsplit // NC
    mesh = plsc.VectorSubcoreMesh(core_axis_name="c", subcore_axis_name="s",
                                  num_cores=NC, num_subcores=NS)

    @functools.partial(
        pl.kernel,
        out_type=jax.ShapeDtypeStruct((nsplit, N_PAD, dh), jnp.float32),
        mesh=mesh,
        compiler_params=pltpu.CompilerParams(use_tc_tiling_on_sc=False,
                                             needs_layout_passes=False),
        scratch_types=[
            pltpu.VMEM((NRB * K, dh), jnp.float32),
            pltpu.VMEM((NLB * K,), jnp.int32),
            pltpu.VMEM((DROWS, B), jnp.int32),
            pltpu.VMEM((NLB * K,), jnp.float32),
            pltpu.VMEM_SHARED((N_PAD, dh), jnp.float32),
            pltpu.SemaphoreType.DMA,
            pltpu.SemaphoreType.DMA,
            pltpu.SemaphoreType.DMA,
        ],
    )
    def mp(sup_hbm, src_hbm, dst_hbm, w_hbm, out_hbm,
           rows_v, src_v, dst_v, w_v, acc_s, sem_in, sem_g, sem_sc):
        c = lax.axis_index("c")
        s = lax.axis_index("s")
        zero = jnp.zeros((L,), jnp.float32)
        ebase = s * EPW

        pltpu.sync_copy(dst_hbm.at[pl.ds(s * DROWS, DROWS)], dst_v)

        def lin_issue(g):
            lb = lax.rem(g, NLB) * K
            off = ebase + g * K
            pltpu.async_copy(src_hbm.at[pl.ds(off, K)], src_v.at[pl.ds(lb, K)], sem_in)
            pltpu.async_copy(w_hbm.at[pl.ds(off, K)], w_v.at[pl.ds(lb, K)], sem_in)

        def lin_drain(g):
            lb = lax.rem(g, NLB) * K
            pltpu.make_async_copy(src_hbm.at[pl.ds(0, K)], src_v.at[pl.ds(lb, K)], sem_in).wait()
            pltpu.make_async_copy(w_hbm.at[pl.ds(0, K)], w_v.at[pl.ds(lb, K)], sem_in).wait()

        def adjust(g, qq):
            lb = lax.rem(g, NLB) * K

            @plsc.parallel_loop(0, K // L, 1, unroll=4)
            def _(j):
                sl = pl.ds(lb + j * L, L)
                src_v[sl] = src_v[sl] * nsplit + qq

        def gather_issue(g, rb):
            lb = lax.rem(g, NLB) * K
            for j in range(NB):
                pltpu.async_copy(sup_hbm.at[src_v.at[pl.ds(lb + j * B, B)]],
                                 rows_v.at[pl.ds(rb * K + j * B, B)], sem_g)

        def gather_drain(rb):
            for j in range(NB):
                pltpu.make_async_copy(sup_hbm.at[pl.ds(0, B)],
                                      rows_v.at[pl.ds(rb * K + j * B, B)], sem_g).wait()

        def scale(g, rb):
            lb = lax.rem(g, NLB) * K
            base = rb * K

            @plsc.parallel_loop(0, K, 1, unroll=2)
            def _(e):
                wv = plsc.load_gather(w_v, [jnp.full((L,), lb + e, jnp.int32)])
                for f in range(dh // L):
                    sl = pl.ds(f * L, L)
                    rows_v[base + e, sl] = rows_v[base + e, sl] * wv

        def scatter_issue(g, rb):
            for j in range(NB):
                pltpu.async_copy(rows_v.at[pl.ds(rb * K + j * B, B)],
                                 acc_s.at[dst_v.at[g * NB + j]], sem_sc, add=True)

        def scatter_drain(rb):
            for j in range(NB):
                pltpu.make_async_copy(rows_v.at[pl.ds(rb * K + j * B, B)],
                                      acc_s.at[pl.ds(0, B)], sem_sc).wait()

        for q in range(npass):
            qq = q * NC + c

            @plsc.parallel_loop(0, ROWS_PER_TILE, 1, unroll=4)
            def _(i):
                for f in range(dh // L):
                    rows_v[i, pl.ds(f * L, L)] = zero
            pltpu.sync_copy(rows_v.at[pl.ds(0, ROWS_PER_TILE)],
                            acc_s.at[pl.ds(s * ROWS_PER_TILE, ROWS_PER_TILE)])
            plsc.subcore_barrier()

            lin_issue(0)
            lin_issue(1)
            lin_issue(2)
            lin_drain(0)
            adjust(0, qq)
            gather_issue(0, 0)

            def chunk_body(gg, _):
                nxt = gg + 1

                @pl.when(nxt < NCH)
                def _():
                    lin_drain(nxt)
                    adjust(nxt, qq)

                    @pl.when(nxt >= NRB)
                    def _():
                        scatter_drain(lax.rem(nxt, NRB))
                    gather_issue(nxt, lax.rem(nxt, NRB))

                    @pl.when(nxt + 2 < NCH)
                    def _():
                        lin_issue(nxt + 2)

                rb = lax.rem(gg, NRB)
                gather_drain(rb)
                scale(gg, rb)
                scatter_issue(gg, rb)
                return 0
            lax.fori_loop(0, NCH, chunk_body, 0)

            for t in range(NRB):
                scatter_drain(t)
            plsc.subcore_barrier()

            pltpu.sync_copy(acc_s.at[pl.ds(s * ROWS_PER_TILE, ROWS_PER_TILE)],
                            out_hbm.at[qq, pl.ds(s * ROWS_PER_TILE, ROWS_PER_TILE)])
            plsc.subcore_barrier()

    return mp(sup_flat, src1d, dst2d, w1d)


def _mm1_body(x_ref, w_ref, o_ref):
    o_ref[...] = jnp.dot(x_ref[...], w_ref[...], preferred_element_type=jnp.float32)


def _mm2_body(a_ref, b_ref, w_ref, o_ref):
    acc = None
    for j in range(a_ref.shape[0]):
        h = jax.nn.relu(a_ref[j] + b_ref[j])
        t = jnp.dot(h, w_ref[j], preferred_element_type=jnp.float32)
        acc = t if acc is None else acc + t
    o_ref[...] = acc


def _fin_body(a_ref, b_ref, o_ref):
    z = jnp.concatenate([a_ref[j] for j in range(a_ref.shape[0])], axis=1) + b_ref[...]
    m = jnp.max(z, axis=1, keepdims=True)
    zs = z - m
    lse = jnp.log(jnp.sum(jnp.exp(zs), axis=1, keepdims=True))
    o_ref[...] = zs - lse


def kernel(x, edge_index, edge_weight, W1, b1, W2, b2):
    n_edges = edge_index.shape[1]
    src = edge_index[0].astype(jnp.int32)
    dst = edge_index[1].astype(jnp.int32)
    pad = E_PAD - n_edges
    src1d = jnp.concatenate([src, jnp.zeros((pad,), jnp.int32)])
    dst2d = jnp.concatenate([dst, jnp.zeros((pad,), jnp.int32)]).reshape(E_PAD // B, B)
    w1d = jnp.concatenate([edge_weight, jnp.zeros((pad,), jnp.float32)])

    support1 = pl.pallas_call(
        _mm1_body,
        out_shape=jax.ShapeDtypeStruct((N_NODES, NHID), jnp.float32),
    )(x, W1)

    ns1 = 8
    d1 = NHID // ns1
    agg1 = _sc_mp(support1.reshape(ns1 * N_NODES, d1), src1d, dst2d, w1d, ns1, d1)

    BN = 2048
    support2 = pl.pallas_call(
        _mm2_body,
        grid=(N_PAD // BN,),
        in_specs=[
            pl.BlockSpec((ns1, BN, d1), lambda i: (0, i, 0)),
            pl.BlockSpec((ns1, 1, d1), lambda i: (0, 0, 0)),
            pl.BlockSpec((ns1, d1, NCLASS), lambda i: (0, 0, 0)),
        ],
        out_specs=pl.BlockSpec((BN, NCLASS), lambda i: (i, 0)),
        out_shape=jax.ShapeDtypeStruct((N_PAD, NCLASS), jnp.float32),
    )(agg1, b1.reshape(ns1, 1, d1), W2.reshape(ns1, d1, NCLASS))

    ns2 = 2
    d2 = NCLASS // ns2
    agg2 = _sc_mp(support2.reshape(ns2 * N_PAD, d2), src1d, dst2d, w1d, ns2, d2)

    out = pl.pallas_call(
        _fin_body,
        out_shape=jax.ShapeDtypeStruct((N_PAD, NCLASS), jnp.float32),
    )(agg2, b2.reshape(1, NCLASS))
    return out[:N_NODES]
